# baseline (speedup 1.0000x reference)
import jax
import jax.numpy as jnp
from jax import lax
from jax.experimental import pallas as pl
from jax.experimental.pallas import tpu as pltpu
from jax.experimental.pallas import tpu_sc as plsc

EPS = 1e-10
_FBLK = 57
_FSC = 57
_FTC = 513 - _FSC
_TC_LANES = 512

_LN2 = 0.6931471805599453
_LOG_COEFFS = (
    5.23940336e-09, 9.99998911e-01, -4.99962245e-01, 3.32818425e-01,
    -2.46356606e-01, 1.84688485e-01, -1.25266614e-01, 6.65124793e-02,
    -2.30382799e-02, 3.75262421e-03,
)


def _fast_log(y):
    b = lax.bitcast_convert_type(y, jnp.int32)
    e = ((b >> 23) - 127).astype(jnp.float32)
    m = lax.bitcast_convert_type((b & 0x007FFFFF) | 0x3F800000, jnp.float32)
    z = m - 1.0
    p = jnp.float32(_LOG_COEFFS[-1])
    for c in _LOG_COEFFS[-2::-1]:
        p = p * z + jnp.float32(c)
    return e * jnp.float32(_LN2) + p


def _tc_body(pred_ref, tar_ref, mask_ref, out_ref, s_acc, m_f32):
    i = pl.program_id(0)
    B = mask_ref.shape[0]
    T = mask_ref.shape[1]

    @pl.when(i == 0)
    def _():
        s_acc[...] = jnp.zeros_like(s_acc)
        m_f32[...] = mask_ref[...].astype(jnp.float32)

    for tc in range(T // _TC_LANES):
        sl = pl.ds(tc * _TC_LANES, _TC_LANES)
        m = m_f32[:, sl]

        def f_body(f, acc):
            p = pred_ref[f, :, sl]
            y = tar_ref[f, :, sl]
            return acc + jnp.abs(p - jnp.log(y + EPS)) * m

        acc = lax.fori_loop(0, _FBLK, f_body,
                            jnp.zeros((B, _TC_LANES), jnp.float32))
        s_acc[:, sl] += acc

    @pl.when(i == pl.num_programs(0) - 1)
    def _():
        s = jnp.sum(s_acc[...]).reshape(1, 1)
        c = jnp.sum(m_f32[...]).reshape(1, 1)
        out_ref[...] = jnp.concatenate([s, c], axis=1)


def _sc_body(pred_hbm, tar_hbm, mask_hbm, out_hbm,
             pb0, pb1, yb0, yb1, mbuf, mf, ovec, sem0, sem1):
    w = lax.axis_index("s") * 2 + lax.axis_index("c")
    tb = w // 16
    tt = w % 16
    rs = pl.ds(tb * 8, 8)
    cs = pl.ds(tt * 128, 128)

    pltpu.sync_copy(mask_hbm.at[rs, cs], mbuf)
    for r in range(8):
        for c in range(8):
            csl = pl.ds(c * 16, 16)
            mf[r, csl] = mbuf[r, csl].astype(jnp.float32)

    def fire(k, pb, yb, sem):
        pltpu.make_async_copy(pred_hbm.at[_FTC + k, rs, cs], pb, sem).start()
        pltpu.make_async_copy(tar_hbm.at[_FTC + k, rs, cs], yb, sem).start()

    def wait(k, pb, yb, sem):
        pltpu.make_async_copy(pred_hbm.at[_FTC + k, rs, cs], pb, sem).wait()
        pltpu.make_async_copy(tar_hbm.at[_FTC + k, rs, cs], yb, sem).wait()

    def plane_sum(pb, yb, acc):
        def rbody(r, a):
            def cbody(c, a2):
                csl = pl.ds(c * 16, 16)
                vp = pb[r, csl]
                vy = yb[r, csl]
                vm = mf[r, csl]
                return a2 + jnp.abs(vp - _fast_log(vy + EPS)) * vm
            return lax.fori_loop(0, 8, cbody, a)
        return lax.fori_loop(0, 8, rbody, acc)

    def plane_sum_into(pb, yb, accr):
        local = plane_sum(pb, yb, jnp.zeros((16,), jnp.float32))
        accr[...] += local

    ovec[...] = jnp.zeros((16,), jnp.float32)
    fire(0, pb0, yb0, sem0)
    fire(1, pb1, yb1, sem1)

    def loop2(k, carry):
        p0 = 2 * k

        wait(p0, pb0, yb0, sem0)
        plane_sum_into(pb0, yb0, ovec)

        @pl.when(p0 + 2 < _FSC)
        def _():
            fire(p0 + 2, pb0, yb0, sem0)

        @pl.when(p0 + 1 < _FSC)
        def _():
            wait(p0 + 1, pb1, yb1, sem1)
            plane_sum_into(pb1, yb1, ovec)

            @pl.when(p0 + 3 < _FSC)
            def _():
                fire(p0 + 3, pb1, yb1, sem1)

        return carry

    lax.fori_loop(0, (_FSC + 1) // 2, loop2, 0)
    pltpu.sync_copy(ovec, out_hbm.at[w])


def kernel(log_predicted, linear_tar, stft_length_masks):
    B, T, F = log_predicted.shape
    pred_t = jnp.transpose(log_predicted, (2, 0, 1))
    tar_t = jnp.transpose(linear_tar, (2, 0, 1))

    tc_out = pl.pallas_call(
        _tc_body,
        grid=(_FTC // _FBLK,),
        in_specs=[
            pl.BlockSpec((_FBLK, B, T), lambda i: (i, 0, 0)),
            pl.BlockSpec((_FBLK, B, T), lambda i: (i, 0, 0)),
            pl.BlockSpec((B, T), lambda i: (0, 0)),
        ],
        out_specs=pl.BlockSpec((1, 2), lambda i: (0, 0)),
        out_shape=jax.ShapeDtypeStruct((1, 2), jnp.float32),
        scratch_shapes=[
            pltpu.VMEM((B, T), jnp.float32),
            pltpu.VMEM((B, T), jnp.float32),
        ],
    )(pred_t, tar_t, stft_length_masks)

    mesh = plsc.VectorSubcoreMesh(core_axis_name="c", subcore_axis_name="s")
    sc_out = pl.kernel(
        _sc_body,
        out_type=jax.ShapeDtypeStruct((32, 16), jnp.float32),
        mesh=mesh,
        compiler_params=pltpu.CompilerParams(use_tc_tiling_on_sc=True),
        scratch_types=[
            pltpu.VMEM((8, 128), jnp.float32),
            pltpu.VMEM((8, 128), jnp.float32),
            pltpu.VMEM((8, 128), jnp.float32),
            pltpu.VMEM((8, 128), jnp.float32),
            pltpu.VMEM((8, 128), jnp.int32),
            pltpu.VMEM((8, 128), jnp.float32),
            pltpu.VMEM((16,), jnp.float32),
            pltpu.SemaphoreType.DMA,
            pltpu.SemaphoreType.DMA,
        ],
    )(pred_t, tar_t, stft_length_masks)

    total = tc_out[0, 0] + jnp.sum(sc_out)
    return total / (tc_out[0, 1] * F)


# speedup vs baseline: 1.7460x; 1.7460x over previous
import jax
import jax.numpy as jnp
from jax.experimental import pallas as pl
from jax.experimental.pallas import tpu as pltpu

EPS = 1e-10
_FBLK = 57
_TC = 1024


def _body(pred_ref, tar_ref, mask_ref, out_ref, s_acc, m_f32):
    i = pl.program_id(0)
    B = mask_ref.shape[0]
    T = mask_ref.shape[1]
    F = pl.num_programs(0) * _FBLK

    @pl.when(i == 0)
    def _():
        s_acc[...] = jnp.zeros_like(s_acc)
        m_f32[...] = mask_ref[...].astype(jnp.float32)

    for tc in range(T // _TC):
        sl = pl.ds(tc * _TC, _TC)
        m = m_f32[:, sl]

        def f_body(f, acc):
            p = pred_ref[f, :, sl]
            y = tar_ref[f, :, sl]
            return acc + jnp.abs(p - jnp.log(y + EPS)) * m

        acc = jax.lax.fori_loop(0, _FBLK, f_body, jnp.zeros((B, _TC), jnp.float32))
        s_acc[:, sl] += acc

    @pl.when(i == pl.num_programs(0) - 1)
    def _():
        out_ref[...] = (jnp.sum(s_acc[...]) / (jnp.sum(m_f32[...]) * F)).reshape(1, 1)


def kernel(log_predicted, linear_tar, stft_length_masks):
    B, T, F = log_predicted.shape
    pred_t = jnp.transpose(log_predicted, (2, 0, 1))
    tar_t = jnp.transpose(linear_tar, (2, 0, 1))

    out = pl.pallas_call(
        _body,
        grid=(F // _FBLK,),
        in_specs=[
            pl.BlockSpec((_FBLK, B, T), lambda i: (i, 0, 0)),
            pl.BlockSpec((_FBLK, B, T), lambda i: (i, 0, 0)),
            pl.BlockSpec((B, T), lambda i: (0, 0)),
        ],
        out_specs=pl.BlockSpec((1, 1), lambda i: (0, 0)),
        out_shape=jax.ShapeDtypeStruct((1, 1), jnp.float32),
        scratch_shapes=[
            pltpu.VMEM((B, T), jnp.float32),
            pltpu.VMEM((B, T), jnp.float32),
        ],
    )(pred_t, tar_t, stft_length_masks)
    return out[0, 0]
